# Initial kernel scaffold; baseline (speedup 1.0000x reference)
import jax
import jax.numpy as jnp
from jax.experimental import pallas as pl


def kernel(node_feats, weight, segment_ids, W1_0, b1_0, Wp_0, bp_0, Wih_0, Whh_0, bih_0, bhh_0, W1_1, b1_1, Wp_1, bp_1, Wih_1, Whh_1, bih_1, bhh_1):
    raise NotImplementedError("write your pallas kernel here")


# speedup vs baseline: 1.4735x; 1.4735x over previous
import functools
import jax
import jax.numpy as jnp
from jax import lax
from jax.experimental import pallas as pl

V = 100000
F = 512
SG = 2048
R = 1024


def _onehot(seg_t):
    cols = lax.broadcasted_iota(jnp.int32, (R, SG), 1)
    return (seg_t == cols).astype(jnp.bfloat16)


def _split(y):
    hi = y.astype(jnp.bfloat16)
    lo = (y - hi.astype(jnp.float32)).astype(jnp.bfloat16)
    return hi, lo


def _pt_dot(p, y):
    hi, lo = _split(y)
    dn = (((0,), (0,)), ((), ()))
    return (lax.dot_general(p, hi, dn, preferred_element_type=jnp.float32) +
            lax.dot_general(p, lo, dn, preferred_element_type=jnp.float32))


def _seg_sum_w_body(x_ref, w_ref, seg_ref, s_ref):
    i = pl.program_id(0)

    @pl.when(i == 0)
    def _():
        s_ref[...] = jnp.zeros_like(s_ref)

    p = _onehot(seg_ref[...])
    s_ref[...] += _pt_dot(p, w_ref[...] * x_ref[...])


def _seg_attn_body(x_ref, seg_ref, qb_ref, w1b_ref, s_ref, den_ref):
    i = pl.program_id(0)

    @pl.when(i == 0)
    def _():
        s_ref[...] = jnp.zeros_like(s_ref)
        den_ref[...] = jnp.zeros_like(den_ref)

    p = _onehot(seg_ref[...])
    x = x_ref[...]
    qh, ql = _split(qb_ref[...])
    qs = (jnp.dot(p, qh, preferred_element_type=jnp.float32) +
          jnp.dot(p, ql, preferred_element_type=jnp.float32))
    xh, xl = _split(x)
    wh, wl = _split(w1b_ref[...])
    c = (jnp.dot(xh, wh, preferred_element_type=jnp.float32) +
         jnp.dot(xh, wl, preferred_element_type=jnp.float32) +
         jnp.dot(xl, wh, preferred_element_type=jnp.float32))
    z = qs + c
    z = jnp.where(z > 0, z, 0.01 * z)
    e = jnp.exp(z)
    den_ref[...] += _pt_dot(p, e)
    s_ref[...] += _pt_dot(p, e * x)


def _q_body(h_ref, w1a_ref, b1_ref, qb_ref):
    h = h_ref[...]
    qb_ref[...] = jnp.sum(jnp.maximum(h, 0.0) * w1a_ref[...], axis=1,
                          keepdims=True) + b1_ref[...]


def _update_body(s_ref, den_ref, h_ref, wp_ref, bp_ref, wih_ref, whh_ref,
                 bih_ref, bhh_ref, w1a_ref, b1_ref, hn_ref, qb_ref):
    den = den_ref[...]
    mask = den > 0
    sn = s_ref[...] * jnp.where(mask, 1.0 / jnp.where(mask, den, 1.0), 0.0)
    g = lax.dot_general(sn, wp_ref[...], (((1,), (1,)), ((), ())),
                        preferred_element_type=jnp.float32, precision=lax.Precision.HIGHEST) + bp_ref[...]
    g = jnp.where(mask, g, 0.0)
    ctx = jnp.where(g > 0, g, jnp.exp(jnp.minimum(g, 0.0)) - 1.0)
    h = h_ref[...]
    gi = lax.dot_general(ctx, wih_ref[...], (((1,), (1,)), ((), ())),
                         preferred_element_type=jnp.float32, precision=lax.Precision.HIGHEST) + bih_ref[...]
    gh = lax.dot_general(h, whh_ref[...], (((1,), (1,)), ((), ())),
                         preferred_element_type=jnp.float32, precision=lax.Precision.HIGHEST) + bhh_ref[...]
    r = jax.nn.sigmoid(gi[:, :F] + gh[:, :F])
    zg = jax.nn.sigmoid(gi[:, F:2 * F] + gh[:, F:2 * F])
    n = jnp.tanh(gi[:, 2 * F:] + r * gh[:, 2 * F:])
    hn = (1.0 - zg) * n + zg * h
    hn_ref[...] = hn
    qb_ref[...] = jnp.sum(jnp.maximum(hn, 0.0) * w1a_ref[...], axis=1,
                          keepdims=True) + b1_ref[...]


def kernel(node_feats, weight, segment_ids, W1_0, b1_0, Wp_0, bp_0, Wih_0,
           Whh_0, bih_0, bhh_0, W1_1, b1_1, Wp_1, bp_1, Wih_1, Whh_1, bih_1,
           bhh_1):
    vp = ((V + R - 1) // R) * R
    nb = vp // R
    pad = vp - V
    x = jnp.pad(node_feats, ((0, pad), (0, 0)))
    w = jnp.pad(weight, (0, pad)).reshape(vp, 1)
    seg = jnp.pad(segment_ids.astype(jnp.int32), (0, pad),
                  constant_values=SG).reshape(vp, 1)

    xspec = pl.BlockSpec((R, F), lambda i: (i, 0))
    vspec = pl.BlockSpec((R, 1), lambda i: (i, 0))
    sspec = pl.BlockSpec((SG, F), lambda i: (0, 0))
    dspec = pl.BlockSpec((SG, 1), lambda i: (0, 0))

    s0 = pl.pallas_call(
        _seg_sum_w_body,
        grid=(nb,),
        in_specs=[xspec, vspec, vspec],
        out_specs=sspec,
        out_shape=jax.ShapeDtypeStruct((SG, F), jnp.float32),
    )(x, w, seg)

    SGB = min(512, SG)
    rowspec = pl.BlockSpec((SGB, F), lambda i: (i, 0))
    rvspec = pl.BlockSpec((SGB, 1), lambda i: (i, 0))
    full = lambda a, b: pl.BlockSpec(a, lambda i: b)

    def q_of(h, w1a, b1):
        return pl.pallas_call(
            _q_body,
            grid=(SG // SGB,),
            in_specs=[rowspec, full((1, F), (0, 0)), full((1, 1), (0, 0))],
            out_specs=rvspec,
            out_shape=jax.ShapeDtypeStruct((SG, 1), jnp.float32),
        )(h, w1a, b1)

    def attn(qb, w1b):
        return pl.pallas_call(
            _seg_attn_body,
            grid=(nb,),
            in_specs=[xspec, vspec, full((SG, 1), (0, 0)),
                      full((F, 1), (0, 0))],
            out_specs=[sspec, dspec],
            out_shape=[jax.ShapeDtypeStruct((SG, F), jnp.float32),
                       jax.ShapeDtypeStruct((SG, 1), jnp.float32)],
        )(x, seg, qb, w1b)

    def update(s, den, h, Wp, bp, Wih, Whh, bih, bhh, w1a_n, b1_n):
        return pl.pallas_call(
            _update_body,
            grid=(SG // SGB,),
            in_specs=[rowspec, rvspec, rowspec,
                      full((F, F), (0, 0)), full((1, F), (0, 0)),
                      full((3 * F, F), (0, 0)), full((3 * F, F), (0, 0)),
                      full((1, 3 * F), (0, 0)), full((1, 3 * F), (0, 0)),
                      full((1, F), (0, 0)), full((1, 1), (0, 0))],
            out_specs=[rowspec, rvspec],
            out_shape=[jax.ShapeDtypeStruct((SG, F), jnp.float32),
                       jax.ShapeDtypeStruct((SG, 1), jnp.float32)],
        )(s, den, h, Wp, bp, Wih, Whh, bih, bhh, w1a_n, b1_n)

    w1a_0 = W1_0[:, :F]
    w1b_0 = W1_0[:, F:].reshape(F, 1)
    w1a_1 = W1_1[:, :F]
    w1b_1 = W1_1[:, F:].reshape(F, 1)
    b1_0r = b1_0.reshape(1, 1)
    b1_1r = b1_1.reshape(1, 1)
    bp_0r = bp_0.reshape(1, F)
    bp_1r = bp_1.reshape(1, F)
    bih_0r = bih_0.reshape(1, 3 * F)
    bhh_0r = bhh_0.reshape(1, 3 * F)
    bih_1r = bih_1.reshape(1, 3 * F)
    bhh_1r = bhh_1.reshape(1, 3 * F)

    qb0 = q_of(s0, w1a_0, b1_0r)
    s_a, den_a = attn(qb0, w1b_0)
    h1, qb1 = update(s_a, den_a, s0, Wp_0, bp_0r, Wih_0, Whh_0, bih_0r,
                     bhh_0r, w1a_1, b1_1r)
    s_b, den_b = attn(qb1, w1b_1)
    h2, _ = update(s_b, den_b, h1, Wp_1, bp_1r, Wih_1, Whh_1, bih_1r,
                   bhh_1r, w1a_1, b1_1r)
    return h2


# speedup vs baseline: 4.5381x; 3.0798x over previous
import functools
import jax
import jax.numpy as jnp
from jax import lax
from jax.experimental import pallas as pl
from jax.experimental.pallas import tpu as pltpu

V = 100000
F = 512
SG = 2048
R = 1024


W = 512


def _onehot(seg_t):
    cols = lax.broadcasted_iota(jnp.int32, (R, SG), 1)
    return (seg_t == cols).astype(jnp.bfloat16)


def _onehot_win(rel):
    cols = lax.broadcasted_iota(jnp.int32, (R, W), 1)
    return (rel == cols).astype(jnp.bfloat16)


def _split(y):
    hi = y.astype(jnp.bfloat16)
    lo = (y - hi.astype(jnp.float32)).astype(jnp.bfloat16)
    return hi, lo


def _pt_dot(p, y):
    hi, lo = _split(y)
    dn = (((0,), (0,)), ((), ()))
    return (lax.dot_general(p, hi, dn, preferred_element_type=jnp.float32) +
            lax.dot_general(p, lo, dn, preferred_element_type=jnp.float32))


def _seg_sum_w_body(first_ref, x_ref, w_ref, seg_ref, s_ref):
    i = pl.program_id(0)

    @pl.when(i == 0)
    def _():
        s_ref[...] = jnp.zeros_like(s_ref)

    seg_t = seg_ref[...]
    lo8 = jnp.minimum((first_ref[0, 0, 0] // 8) * 8, SG - W)
    rel = seg_t - lo8
    p = _onehot_win(rel)
    y = w_ref[...] * x_ref[...]
    s_ref[pl.ds(lo8, W), :] += _pt_dot(p, y)

    @pl.when(jnp.max(rel) >= W)
    def _():
        pf = _onehot(seg_t) * (rel >= W).astype(jnp.bfloat16)
        s_ref[...] += _pt_dot(pf, y)


def _seg_attn_body(first_ref, x_ref, seg_ref, qb_ref, w1b_ref, s_ref,
                   den_ref, qs_ref):
    i = pl.program_id(0)

    @pl.when(i == 0)
    def _():
        s_ref[...] = jnp.zeros_like(s_ref)
        den_ref[...] = jnp.zeros_like(den_ref)

    seg_t = seg_ref[...]
    lo8 = jnp.minimum((first_ref[0, 0, 0] // 8) * 8, SG - W)
    rel = seg_t - lo8
    p = _onehot_win(rel)
    x = x_ref[...]
    overflow = jnp.max(rel) >= W

    qh, ql = _split(qb_ref[pl.ds(lo8, W), :])
    qs_ref[...] = (jnp.dot(p, qh, preferred_element_type=jnp.float32) +
                   jnp.dot(p, ql, preferred_element_type=jnp.float32))

    @pl.when(overflow)
    def _():
        pf = _onehot(seg_t) * (rel >= W).astype(jnp.bfloat16)
        qfh, qfl = _split(qb_ref[...])
        qs_ref[...] += (jnp.dot(pf, qfh, preferred_element_type=jnp.float32)
                        + jnp.dot(pf, qfl,
                                  preferred_element_type=jnp.float32))

    xh, xl = _split(x)
    wh, wl = _split(w1b_ref[...])
    c = (jnp.dot(xh, wh, preferred_element_type=jnp.float32) +
         jnp.dot(xh, wl, preferred_element_type=jnp.float32) +
         jnp.dot(xl, wh, preferred_element_type=jnp.float32))
    z = qs_ref[...] + c
    z = jnp.where(z > 0, z, 0.01 * z)
    e = jnp.exp(z)
    den_ref[pl.ds(lo8, W), :] += _pt_dot(p, e)
    s_ref[pl.ds(lo8, W), :] += _pt_dot(p, e * x)

    @pl.when(overflow)
    def _():
        pf = _onehot(seg_t) * (rel >= W).astype(jnp.bfloat16)
        den_ref[...] += _pt_dot(pf, e)
        s_ref[...] += _pt_dot(pf, e * x)


def _q_body(h_ref, w1a_ref, b1_ref, qb_ref):
    h = h_ref[...]
    qb_ref[...] = jnp.sum(jnp.maximum(h, 0.0) * w1a_ref[...], axis=1,
                          keepdims=True) + b1_ref[...]


def _update_body(s_ref, den_ref, h_ref, wp_ref, bp_ref, wih_ref, whh_ref,
                 bih_ref, bhh_ref, w1a_ref, b1_ref, hn_ref, qb_ref):
    den = den_ref[...]
    mask = den > 0
    sn = s_ref[...] * jnp.where(mask, 1.0 / jnp.where(mask, den, 1.0), 0.0)
    g = lax.dot_general(sn, wp_ref[...], (((1,), (1,)), ((), ())),
                        preferred_element_type=jnp.float32, precision=lax.Precision.HIGHEST) + bp_ref[...]
    g = jnp.where(mask, g, 0.0)
    ctx = jnp.where(g > 0, g, jnp.exp(jnp.minimum(g, 0.0)) - 1.0)
    h = h_ref[...]
    gi = lax.dot_general(ctx, wih_ref[...], (((1,), (1,)), ((), ())),
                         preferred_element_type=jnp.float32, precision=lax.Precision.HIGHEST) + bih_ref[...]
    gh = lax.dot_general(h, whh_ref[...], (((1,), (1,)), ((), ())),
                         preferred_element_type=jnp.float32, precision=lax.Precision.HIGHEST) + bhh_ref[...]
    r = jax.nn.sigmoid(gi[:, :F] + gh[:, :F])
    zg = jax.nn.sigmoid(gi[:, F:2 * F] + gh[:, F:2 * F])
    n = jnp.tanh(gi[:, 2 * F:] + r * gh[:, 2 * F:])
    hn = (1.0 - zg) * n + zg * h
    hn_ref[...] = hn
    qb_ref[...] = jnp.sum(jnp.maximum(hn, 0.0) * w1a_ref[...], axis=1,
                          keepdims=True) + b1_ref[...]


def kernel(node_feats, weight, segment_ids, W1_0, b1_0, Wp_0, bp_0, Wih_0,
           Whh_0, bih_0, bhh_0, W1_1, b1_1, Wp_1, bp_1, Wih_1, Whh_1, bih_1,
           bhh_1):
    vp = ((V + R - 1) // R) * R
    nb = vp // R
    pad = vp - V
    x = jnp.pad(node_feats, ((0, pad), (0, 0)))
    w = jnp.pad(weight, (0, pad)).reshape(vp, 1)
    seg = jnp.pad(segment_ids.astype(jnp.int32), (0, pad),
                  constant_values=SG).reshape(vp, 1)

    xspec = pl.BlockSpec((R, F), lambda i: (i, 0))
    vspec = pl.BlockSpec((R, 1), lambda i: (i, 0))
    sspec = pl.BlockSpec((SG, F), lambda i: (0, 0))
    dspec = pl.BlockSpec((SG, 1), lambda i: (0, 0))

    seg_first = seg[::R].reshape(nb, 1, 1)
    fspec = pl.BlockSpec((1, 1, 1), lambda i: (i, 0, 0),
                         memory_space=pltpu.SMEM)

    s0 = pl.pallas_call(
        _seg_sum_w_body,
        grid=(nb,),
        in_specs=[fspec, xspec, vspec, vspec],
        out_specs=sspec,
        out_shape=jax.ShapeDtypeStruct((SG, F), jnp.float32),
    )(seg_first, x, w, seg)

    SGB = min(512, SG)
    rowspec = pl.BlockSpec((SGB, F), lambda i: (i, 0))
    rvspec = pl.BlockSpec((SGB, 1), lambda i: (i, 0))
    full = lambda a, b: pl.BlockSpec(a, lambda i: b)

    def q_of(h, w1a, b1):
        return pl.pallas_call(
            _q_body,
            grid=(SG // SGB,),
            in_specs=[rowspec, full((1, F), (0, 0)), full((1, 1), (0, 0))],
            out_specs=rvspec,
            out_shape=jax.ShapeDtypeStruct((SG, 1), jnp.float32),
        )(h, w1a, b1)

    def attn(qb, w1b):
        return pl.pallas_call(
            _seg_attn_body,
            grid=(nb,),
            in_specs=[fspec, xspec, vspec, full((SG, 1), (0, 0)),
                      full((F, 1), (0, 0))],
            out_specs=[sspec, dspec],
            out_shape=[jax.ShapeDtypeStruct((SG, F), jnp.float32),
                       jax.ShapeDtypeStruct((SG, 1), jnp.float32)],
            scratch_shapes=[pltpu.VMEM((R, 1), jnp.float32)],
        )(seg_first, x, seg, qb, w1b)

    def update(s, den, h, Wp, bp, Wih, Whh, bih, bhh, w1a_n, b1_n):
        return pl.pallas_call(
            _update_body,
            grid=(SG // SGB,),
            in_specs=[rowspec, rvspec, rowspec,
                      full((F, F), (0, 0)), full((1, F), (0, 0)),
                      full((3 * F, F), (0, 0)), full((3 * F, F), (0, 0)),
                      full((1, 3 * F), (0, 0)), full((1, 3 * F), (0, 0)),
                      full((1, F), (0, 0)), full((1, 1), (0, 0))],
            out_specs=[rowspec, rvspec],
            out_shape=[jax.ShapeDtypeStruct((SG, F), jnp.float32),
                       jax.ShapeDtypeStruct((SG, 1), jnp.float32)],
        )(s, den, h, Wp, bp, Wih, Whh, bih, bhh, w1a_n, b1_n)

    w1a_0 = W1_0[:, :F]
    w1b_0 = W1_0[:, F:].reshape(F, 1)
    w1a_1 = W1_1[:, :F]
    w1b_1 = W1_1[:, F:].reshape(F, 1)
    b1_0r = b1_0.reshape(1, 1)
    b1_1r = b1_1.reshape(1, 1)
    bp_0r = bp_0.reshape(1, F)
    bp_1r = bp_1.reshape(1, F)
    bih_0r = bih_0.reshape(1, 3 * F)
    bhh_0r = bhh_0.reshape(1, 3 * F)
    bih_1r = bih_1.reshape(1, 3 * F)
    bhh_1r = bhh_1.reshape(1, 3 * F)

    qb0 = q_of(s0, w1a_0, b1_0r)
    s_a, den_a = attn(qb0, w1b_0)
    h1, qb1 = update(s_a, den_a, s0, Wp_0, bp_0r, Wih_0, Whh_0, bih_0r,
                     bhh_0r, w1a_1, b1_1r)
    s_b, den_b = attn(qb1, w1b_1)
    h2, _ = update(s_b, den_b, h1, Wp_1, bp_1r, Wih_1, Whh_1, bih_1r,
                   bhh_1r, w1a_1, b1_1r)
    return h2


# speedup vs baseline: 4.8383x; 1.0662x over previous
import functools
import jax
import jax.numpy as jnp
from jax import lax
from jax.experimental import pallas as pl
from jax.experimental.pallas import tpu as pltpu

V = 100000
F = 512
SG = 2048
R = 1024


W = 512


def _onehot(seg_t):
    cols = lax.broadcasted_iota(jnp.int32, (R, SG), 1)
    return (seg_t == cols).astype(jnp.bfloat16)


def _onehot_win(rel):
    cols = lax.broadcasted_iota(jnp.int32, (R, W), 1)
    return (rel == cols).astype(jnp.bfloat16)


def _split(y):
    hi = y.astype(jnp.bfloat16)
    lo = (y - hi.astype(jnp.float32)).astype(jnp.bfloat16)
    return hi, lo


def _pt_dot(p, y):
    hi, lo = _split(y)
    dn = (((0,), (0,)), ((), ()))
    return (lax.dot_general(p, hi, dn, preferred_element_type=jnp.float32) +
            lax.dot_general(p, lo, dn, preferred_element_type=jnp.float32))


def _seg_sum_w_body(first_ref, x_ref, w_ref, seg_ref, w1b_ref, s_ref,
                    cc_ref):
    i = pl.program_id(0)

    @pl.when(i == 0)
    def _():
        s_ref[...] = jnp.zeros_like(s_ref)

    seg_t = seg_ref[...]
    lo8 = jnp.minimum((first_ref[0, 0, 0] // 8) * 8, SG - W)
    rel = seg_t - lo8
    p = _onehot_win(rel)
    x = x_ref[...]
    xh, xl = _split(x)
    wbh, wbl = _split(w1b_ref[...])
    cc_ref[...] = (jnp.dot(xh, wbh, preferred_element_type=jnp.float32) +
                   jnp.dot(xh, wbl, preferred_element_type=jnp.float32) +
                   jnp.dot(xl, wbh, preferred_element_type=jnp.float32))
    y = w_ref[...] * x
    s_ref[pl.ds(lo8, W), :] += _pt_dot(p, y)

    @pl.when(jnp.max(rel) >= W)
    def _():
        pf = _onehot(seg_t) * (rel >= W).astype(jnp.bfloat16)
        s_ref[...] += _pt_dot(pf, y)


def _seg_attn_body(first_ref, x_ref, seg_ref, qb_ref, c_ref, s_ref,
                   den_ref, qs_ref):
    i = pl.program_id(0)

    @pl.when(i == 0)
    def _():
        s_ref[...] = jnp.zeros_like(s_ref)
        den_ref[...] = jnp.zeros_like(den_ref)

    seg_t = seg_ref[...]
    lo8 = jnp.minimum((first_ref[0, 0, 0] // 8) * 8, SG - W)
    rel = seg_t - lo8
    p = _onehot_win(rel)
    x = x_ref[...]
    overflow = jnp.max(rel) >= W

    qh, ql = _split(qb_ref[pl.ds(lo8, W), :])
    qs_ref[...] = (jnp.dot(p, qh, preferred_element_type=jnp.float32) +
                   jnp.dot(p, ql, preferred_element_type=jnp.float32))

    @pl.when(overflow)
    def _():
        pf = _onehot(seg_t) * (rel >= W).astype(jnp.bfloat16)
        qfh, qfl = _split(qb_ref[...])
        qs_ref[...] += (jnp.dot(pf, qfh, preferred_element_type=jnp.float32)
                        + jnp.dot(pf, qfl,
                                  preferred_element_type=jnp.float32))

    z = qs_ref[...] + c_ref[...]
    z = jnp.where(z > 0, z, 0.01 * z)
    e = jnp.exp(z)
    den_ref[pl.ds(lo8, W), :] += _pt_dot(p, e)
    s_ref[pl.ds(lo8, W), :] += _pt_dot(p, e * x)

    @pl.when(overflow)
    def _():
        pf = _onehot(seg_t) * (rel >= W).astype(jnp.bfloat16)
        den_ref[...] += _pt_dot(pf, e)
        s_ref[...] += _pt_dot(pf, e * x)


def _q_body(h_ref, w1a_ref, b1_ref, qb_ref):
    h = h_ref[...]
    qb_ref[...] = jnp.sum(jnp.maximum(h, 0.0) * w1a_ref[...], axis=1,
                          keepdims=True) + b1_ref[...]


def _dot3(a, b):
    ah, al = _split(a)
    bh, bl = _split(b)
    dn = (((1,), (1,)), ((), ()))
    return (lax.dot_general(ah, bh, dn, preferred_element_type=jnp.float32) +
            lax.dot_general(ah, bl, dn, preferred_element_type=jnp.float32) +
            lax.dot_general(al, bh, dn, preferred_element_type=jnp.float32))


def _update_body(s_ref, den_ref, h_ref, wp_ref, bp_ref, wih_ref, whh_ref,
                 bih_ref, bhh_ref, w1a_ref, b1_ref, hn_ref, qb_ref):
    den = den_ref[...]
    mask = den > 0
    sn = s_ref[...] * jnp.where(mask, 1.0 / jnp.where(mask, den, 1.0), 0.0)
    g = _dot3(sn, wp_ref[...]) + bp_ref[...]
    g = jnp.where(mask, g, 0.0)
    ctx = jnp.where(g > 0, g, jnp.exp(jnp.minimum(g, 0.0)) - 1.0)
    h = h_ref[...]
    gi = _dot3(ctx, wih_ref[...]) + bih_ref[...]
    gh = _dot3(h, whh_ref[...]) + bhh_ref[...]
    r = jax.nn.sigmoid(gi[:, :F] + gh[:, :F])
    zg = jax.nn.sigmoid(gi[:, F:2 * F] + gh[:, F:2 * F])
    n = jnp.tanh(gi[:, 2 * F:] + r * gh[:, 2 * F:])
    hn = (1.0 - zg) * n + zg * h
    hn_ref[...] = hn
    qb_ref[...] = jnp.sum(jnp.maximum(hn, 0.0) * w1a_ref[...], axis=1,
                          keepdims=True) + b1_ref[...]


def kernel(node_feats, weight, segment_ids, W1_0, b1_0, Wp_0, bp_0, Wih_0,
           Whh_0, bih_0, bhh_0, W1_1, b1_1, Wp_1, bp_1, Wih_1, Whh_1, bih_1,
           bhh_1):
    vp = ((V + R - 1) // R) * R
    nb = vp // R
    pad = vp - V
    x = jnp.pad(node_feats, ((0, pad), (0, 0)))
    w = jnp.pad(weight, (0, pad)).reshape(vp, 1)
    seg = jnp.pad(segment_ids.astype(jnp.int32), (0, pad),
                  constant_values=SG).reshape(vp, 1)

    xspec = pl.BlockSpec((R, F), lambda i: (i, 0))
    vspec = pl.BlockSpec((R, 1), lambda i: (i, 0))
    sspec = pl.BlockSpec((SG, F), lambda i: (0, 0))
    dspec = pl.BlockSpec((SG, 1), lambda i: (0, 0))

    seg_first = seg[::R].reshape(nb, 1, 1)
    fspec = pl.BlockSpec((1, 1, 1), lambda i: (i, 0, 0),
                         memory_space=pltpu.SMEM)

    w1b_both = jnp.concatenate([W1_0[:, F:].reshape(F, 1),
                                W1_1[:, F:].reshape(F, 1)], axis=1)
    s0, cc = pl.pallas_call(
        _seg_sum_w_body,
        grid=(nb,),
        in_specs=[fspec, xspec, vspec, vspec,
                  pl.BlockSpec((F, 2), lambda i: (0, 0))],
        out_specs=[sspec, pl.BlockSpec((R, 2), lambda i: (i, 0))],
        out_shape=[jax.ShapeDtypeStruct((SG, F), jnp.float32),
                   jax.ShapeDtypeStruct((vp, 2), jnp.float32)],
    )(seg_first, x, w, seg, w1b_both)
    c0 = cc[:, 0:1]
    c1 = cc[:, 1:2]

    SGB = min(512, SG)
    rowspec = pl.BlockSpec((SGB, F), lambda i: (i, 0))
    rvspec = pl.BlockSpec((SGB, 1), lambda i: (i, 0))
    full = lambda a, b: pl.BlockSpec(a, lambda i: b)

    def q_of(h, w1a, b1):
        return pl.pallas_call(
            _q_body,
            grid=(SG // SGB,),
            in_specs=[rowspec, full((1, F), (0, 0)), full((1, 1), (0, 0))],
            out_specs=rvspec,
            out_shape=jax.ShapeDtypeStruct((SG, 1), jnp.float32),
        )(h, w1a, b1)

    def attn(qb, cv):
        return pl.pallas_call(
            _seg_attn_body,
            grid=(nb,),
            in_specs=[fspec, xspec, vspec, full((SG, 1), (0, 0)),
                      vspec],
            out_specs=[sspec, dspec],
            out_shape=[jax.ShapeDtypeStruct((SG, F), jnp.float32),
                       jax.ShapeDtypeStruct((SG, 1), jnp.float32)],
            scratch_shapes=[pltpu.VMEM((R, 1), jnp.float32)],
        )(seg_first, x, seg, qb, cv)

    def update(s, den, h, Wp, bp, Wih, Whh, bih, bhh, w1a_n, b1_n):
        return pl.pallas_call(
            _update_body,
            grid=(SG // SGB,),
            in_specs=[rowspec, rvspec, rowspec,
                      full((F, F), (0, 0)), full((1, F), (0, 0)),
                      full((3 * F, F), (0, 0)), full((3 * F, F), (0, 0)),
                      full((1, 3 * F), (0, 0)), full((1, 3 * F), (0, 0)),
                      full((1, F), (0, 0)), full((1, 1), (0, 0))],
            out_specs=[rowspec, rvspec],
            out_shape=[jax.ShapeDtypeStruct((SG, F), jnp.float32),
                       jax.ShapeDtypeStruct((SG, 1), jnp.float32)],
        )(s, den, h, Wp, bp, Wih, Whh, bih, bhh, w1a_n, b1_n)

    w1a_0 = W1_0[:, :F]
    w1b_0 = W1_0[:, F:].reshape(F, 1)
    w1a_1 = W1_1[:, :F]
    w1b_1 = W1_1[:, F:].reshape(F, 1)
    b1_0r = b1_0.reshape(1, 1)
    b1_1r = b1_1.reshape(1, 1)
    bp_0r = bp_0.reshape(1, F)
    bp_1r = bp_1.reshape(1, F)
    bih_0r = bih_0.reshape(1, 3 * F)
    bhh_0r = bhh_0.reshape(1, 3 * F)
    bih_1r = bih_1.reshape(1, 3 * F)
    bhh_1r = bhh_1.reshape(1, 3 * F)

    qb0 = q_of(s0, w1a_0, b1_0r)
    s_a, den_a = attn(qb0, c0)
    h1, qb1 = update(s_a, den_a, s0, Wp_0, bp_0r, Wih_0, Whh_0, bih_0r,
                     bhh_0r, w1a_1, b1_1r)
    s_b, den_b = attn(qb1, c1)
    h2, _ = update(s_b, den_b, h1, Wp_1, bp_1r, Wih_1, Whh_1, bih_1r,
                   bhh_1r, w1a_1, b1_1r)
    return h2


# speedup vs baseline: 4.9324x; 1.0194x over previous
import functools
import jax
import jax.numpy as jnp
from jax import lax
from jax.experimental import pallas as pl
from jax.experimental.pallas import tpu as pltpu

V = 100000
F = 512
SG = 2048
R = 2048


W = 512


def _onehot(seg_t):
    cols = lax.broadcasted_iota(jnp.int32, (R, SG), 1)
    return (seg_t == cols).astype(jnp.bfloat16)


def _onehot_win(rel):
    cols = lax.broadcasted_iota(jnp.int32, (R, W), 1)
    return (rel == cols).astype(jnp.bfloat16)


def _split(y):
    hi = y.astype(jnp.bfloat16)
    lo = (y - hi.astype(jnp.float32)).astype(jnp.bfloat16)
    return hi, lo


def _pt_dot(p, y):
    hi, lo = _split(y)
    dn = (((0,), (0,)), ((), ()))
    return (lax.dot_general(p, hi, dn, preferred_element_type=jnp.float32) +
            lax.dot_general(p, lo, dn, preferred_element_type=jnp.float32))


def _seg_sum_w_body(first_ref, x_ref, w_ref, seg_ref, w1b_ref, s_ref,
                    cc_ref):
    i = pl.program_id(0)

    @pl.when(i == 0)
    def _():
        s_ref[...] = jnp.zeros_like(s_ref)

    seg_t = seg_ref[...]
    lo8 = jnp.minimum((first_ref[0, 0, 0] // 8) * 8, SG - W)
    rel = seg_t - lo8
    p = _onehot_win(rel)
    x = x_ref[...]
    xh, xl = _split(x)
    wbh, wbl = _split(w1b_ref[...])
    cc_ref[...] = (jnp.dot(xh, wbh, preferred_element_type=jnp.float32) +
                   jnp.dot(xh, wbl, preferred_element_type=jnp.float32) +
                   jnp.dot(xl, wbh, preferred_element_type=jnp.float32))
    y = w_ref[...] * x
    s_ref[pl.ds(lo8, W), :] += _pt_dot(p, y)

    @pl.when(jnp.max(rel) >= W)
    def _():
        pf = _onehot(seg_t) * (rel >= W).astype(jnp.bfloat16)
        s_ref[...] += _pt_dot(pf, y)


def _seg_attn_body(first_ref, x_ref, seg_ref, qb_ref, c_ref, s_ref,
                   den_ref, qs_ref):
    i = pl.program_id(0)

    @pl.when(i == 0)
    def _():
        s_ref[...] = jnp.zeros_like(s_ref)
        den_ref[...] = jnp.zeros_like(den_ref)

    seg_t = seg_ref[...]
    lo8 = jnp.minimum((first_ref[0, 0, 0] // 8) * 8, SG - W)
    rel = seg_t - lo8
    p = _onehot_win(rel)
    x = x_ref[...]
    overflow = jnp.max(rel) >= W

    qh, ql = _split(qb_ref[pl.ds(lo8, W), :])
    qs_ref[...] = (jnp.dot(p, qh, preferred_element_type=jnp.float32) +
                   jnp.dot(p, ql, preferred_element_type=jnp.float32))

    @pl.when(overflow)
    def _():
        pf = _onehot(seg_t) * (rel >= W).astype(jnp.bfloat16)
        qfh, qfl = _split(qb_ref[...])
        qs_ref[...] += (jnp.dot(pf, qfh, preferred_element_type=jnp.float32)
                        + jnp.dot(pf, qfl,
                                  preferred_element_type=jnp.float32))

    z = qs_ref[...] + c_ref[...]
    z = jnp.where(z > 0, z, 0.01 * z)
    e = jnp.exp(z)
    den_ref[pl.ds(lo8, W), :] += _pt_dot(p, e)
    s_ref[pl.ds(lo8, W), :] += _pt_dot(p, e * x)

    @pl.when(overflow)
    def _():
        pf = _onehot(seg_t) * (rel >= W).astype(jnp.bfloat16)
        den_ref[...] += _pt_dot(pf, e)
        s_ref[...] += _pt_dot(pf, e * x)


def _q_body(h_ref, w1a_ref, b1_ref, qb_ref):
    h = h_ref[...]
    qb_ref[...] = jnp.sum(jnp.maximum(h, 0.0) * w1a_ref[...], axis=1,
                          keepdims=True) + b1_ref[...]


def _dot3(a, b):
    ah, al = _split(a)
    bh, bl = _split(b)
    dn = (((1,), (1,)), ((), ()))
    return (lax.dot_general(ah, bh, dn, preferred_element_type=jnp.float32) +
            lax.dot_general(ah, bl, dn, preferred_element_type=jnp.float32) +
            lax.dot_general(al, bh, dn, preferred_element_type=jnp.float32))


def _update_body(s_ref, den_ref, h_ref, wp_ref, bp_ref, wih_ref, whh_ref,
                 bih_ref, bhh_ref, w1a_ref, b1_ref, hn_ref, qb_ref):
    den = den_ref[...]
    mask = den > 0
    sn = s_ref[...] * jnp.where(mask, 1.0 / jnp.where(mask, den, 1.0), 0.0)
    g = _dot3(sn, wp_ref[...]) + bp_ref[...]
    g = jnp.where(mask, g, 0.0)
    ctx = jnp.where(g > 0, g, jnp.exp(jnp.minimum(g, 0.0)) - 1.0)
    h = h_ref[...]
    gi = _dot3(ctx, wih_ref[...]) + bih_ref[...]
    gh = _dot3(h, whh_ref[...]) + bhh_ref[...]
    r = jax.nn.sigmoid(gi[:, :F] + gh[:, :F])
    zg = jax.nn.sigmoid(gi[:, F:2 * F] + gh[:, F:2 * F])
    n = jnp.tanh(gi[:, 2 * F:] + r * gh[:, 2 * F:])
    hn = (1.0 - zg) * n + zg * h
    hn_ref[...] = hn
    qb_ref[...] = jnp.sum(jnp.maximum(hn, 0.0) * w1a_ref[...], axis=1,
                          keepdims=True) + b1_ref[...]


def kernel(node_feats, weight, segment_ids, W1_0, b1_0, Wp_0, bp_0, Wih_0,
           Whh_0, bih_0, bhh_0, W1_1, b1_1, Wp_1, bp_1, Wih_1, Whh_1, bih_1,
           bhh_1):
    vp = ((V + R - 1) // R) * R
    nb = vp // R
    pad = vp - V
    x = jnp.pad(node_feats, ((0, pad), (0, 0)))
    w = jnp.pad(weight, (0, pad)).reshape(vp, 1)
    seg = jnp.pad(segment_ids.astype(jnp.int32), (0, pad),
                  constant_values=SG).reshape(vp, 1)

    xspec = pl.BlockSpec((R, F), lambda i: (i, 0))
    vspec = pl.BlockSpec((R, 1), lambda i: (i, 0))
    sspec = pl.BlockSpec((SG, F), lambda i: (0, 0))
    dspec = pl.BlockSpec((SG, 1), lambda i: (0, 0))

    seg_first = seg[::R].reshape(nb, 1, 1)
    fspec = pl.BlockSpec((1, 1, 1), lambda i: (i, 0, 0),
                         memory_space=pltpu.SMEM)

    w1b_both = jnp.concatenate([W1_0[:, F:].reshape(F, 1),
                                W1_1[:, F:].reshape(F, 1)], axis=1)
    s0, cc = pl.pallas_call(
        _seg_sum_w_body,
        grid=(nb,),
        in_specs=[fspec, xspec, vspec, vspec,
                  pl.BlockSpec((F, 2), lambda i: (0, 0))],
        out_specs=[sspec, pl.BlockSpec((R, 2), lambda i: (i, 0))],
        out_shape=[jax.ShapeDtypeStruct((SG, F), jnp.float32),
                   jax.ShapeDtypeStruct((vp, 2), jnp.float32)],
    )(seg_first, x, w, seg, w1b_both)
    c0 = cc[:, 0:1]
    c1 = cc[:, 1:2]

    SGB = min(512, SG)
    rowspec = pl.BlockSpec((SGB, F), lambda i: (i, 0))
    rvspec = pl.BlockSpec((SGB, 1), lambda i: (i, 0))
    full = lambda a, b: pl.BlockSpec(a, lambda i: b)

    def q_of(h, w1a, b1):
        return pl.pallas_call(
            _q_body,
            grid=(SG // SGB,),
            in_specs=[rowspec, full((1, F), (0, 0)), full((1, 1), (0, 0))],
            out_specs=rvspec,
            out_shape=jax.ShapeDtypeStruct((SG, 1), jnp.float32),
        )(h, w1a, b1)

    def attn(qb, cv):
        return pl.pallas_call(
            _seg_attn_body,
            grid=(nb,),
            in_specs=[fspec, xspec, vspec, full((SG, 1), (0, 0)),
                      vspec],
            out_specs=[sspec, dspec],
            out_shape=[jax.ShapeDtypeStruct((SG, F), jnp.float32),
                       jax.ShapeDtypeStruct((SG, 1), jnp.float32)],
            scratch_shapes=[pltpu.VMEM((R, 1), jnp.float32)],
        )(seg_first, x, seg, qb, cv)

    def update(s, den, h, Wp, bp, Wih, Whh, bih, bhh, w1a_n, b1_n):
        return pl.pallas_call(
            _update_body,
            grid=(SG // SGB,),
            in_specs=[rowspec, rvspec, rowspec,
                      full((F, F), (0, 0)), full((1, F), (0, 0)),
                      full((3 * F, F), (0, 0)), full((3 * F, F), (0, 0)),
                      full((1, 3 * F), (0, 0)), full((1, 3 * F), (0, 0)),
                      full((1, F), (0, 0)), full((1, 1), (0, 0))],
            out_specs=[rowspec, rvspec],
            out_shape=[jax.ShapeDtypeStruct((SG, F), jnp.float32),
                       jax.ShapeDtypeStruct((SG, 1), jnp.float32)],
        )(s, den, h, Wp, bp, Wih, Whh, bih, bhh, w1a_n, b1_n)

    w1a_0 = W1_0[:, :F]
    w1b_0 = W1_0[:, F:].reshape(F, 1)
    w1a_1 = W1_1[:, :F]
    w1b_1 = W1_1[:, F:].reshape(F, 1)
    b1_0r = b1_0.reshape(1, 1)
    b1_1r = b1_1.reshape(1, 1)
    bp_0r = bp_0.reshape(1, F)
    bp_1r = bp_1.reshape(1, F)
    bih_0r = bih_0.reshape(1, 3 * F)
    bhh_0r = bhh_0.reshape(1, 3 * F)
    bih_1r = bih_1.reshape(1, 3 * F)
    bhh_1r = bhh_1.reshape(1, 3 * F)

    qb0 = q_of(s0, w1a_0, b1_0r)
    s_a, den_a = attn(qb0, c0)
    h1, qb1 = update(s_a, den_a, s0, Wp_0, bp_0r, Wih_0, Whh_0, bih_0r,
                     bhh_0r, w1a_1, b1_1r)
    s_b, den_b = attn(qb1, c1)
    h2, _ = update(s_b, den_b, h1, Wp_1, bp_1r, Wih_1, Whh_1, bih_1r,
                   bhh_1r, w1a_1, b1_1r)
    return h2


# speedup vs baseline: 6.3890x; 1.2953x over previous
import functools
import jax
import jax.numpy as jnp
from jax import lax
from jax.experimental import pallas as pl
from jax.experimental.pallas import tpu as pltpu

V = 100000
F = 512
SG = 2048
R = 2048


W = 256


def _onehot(seg_t):
    cols = lax.broadcasted_iota(jnp.int32, (R, SG), 1)
    return (seg_t == cols).astype(jnp.bfloat16)


def _onehot_win(rel):
    cols = lax.broadcasted_iota(jnp.int32, (R, W), 1)
    return (rel == cols).astype(jnp.bfloat16)


def _split(y):
    hi = y.astype(jnp.bfloat16)
    lo = (y - hi.astype(jnp.float32)).astype(jnp.bfloat16)
    return hi, lo


def _pt_dot(p, y):
    hi, lo = _split(y)
    dn = (((0,), (0,)), ((), ()))
    return (lax.dot_general(p, hi, dn, preferred_element_type=jnp.float32) +
            lax.dot_general(p, lo, dn, preferred_element_type=jnp.float32))


def _seg_sum_w_body(first_ref, x_ref, w_ref, seg_ref, w1b_ref, s_ref,
                    cc_ref):
    i = pl.program_id(0)

    @pl.when(i == 0)
    def _():
        s_ref[...] = jnp.zeros_like(s_ref)

    seg_t = seg_ref[...]
    lo8 = jnp.minimum((first_ref[0, 0, 0] // 8) * 8, SG - W)
    rel = seg_t - lo8
    p = _onehot_win(rel)
    x = x_ref[...]
    xh, xl = _split(x)
    wbh, wbl = _split(w1b_ref[...])
    cc_ref[...] = (jnp.dot(xh, wbh, preferred_element_type=jnp.float32) +
                   jnp.dot(xh, wbl, preferred_element_type=jnp.float32) +
                   jnp.dot(xl, wbh, preferred_element_type=jnp.float32))
    y = w_ref[...] * x
    s_ref[pl.ds(lo8, W), :] += _pt_dot(p, y)

    @pl.when(jnp.max(rel) >= W)
    def _():
        pf = _onehot(seg_t) * (rel >= W).astype(jnp.bfloat16)
        s_ref[...] += _pt_dot(pf, y)


def _seg_attn_body(first_ref, x_ref, seg_ref, qb_ref, c_ref, s_ref,
                   den_ref, qs_ref):
    i = pl.program_id(0)

    @pl.when(i == 0)
    def _():
        s_ref[...] = jnp.zeros_like(s_ref)
        den_ref[...] = jnp.zeros_like(den_ref)

    seg_t = seg_ref[...]
    lo8 = jnp.minimum((first_ref[0, 0, 0] // 8) * 8, SG - W)
    rel = seg_t - lo8
    p = _onehot_win(rel)
    x = x_ref[...]
    overflow = jnp.max(rel) >= W

    qh, ql = _split(qb_ref[pl.ds(lo8, W), :])
    qs_ref[...] = (jnp.dot(p, qh, preferred_element_type=jnp.float32) +
                   jnp.dot(p, ql, preferred_element_type=jnp.float32))

    @pl.when(overflow)
    def _():
        pf = _onehot(seg_t) * (rel >= W).astype(jnp.bfloat16)
        qfh, qfl = _split(qb_ref[...])
        qs_ref[...] += (jnp.dot(pf, qfh, preferred_element_type=jnp.float32)
                        + jnp.dot(pf, qfl,
                                  preferred_element_type=jnp.float32))

    z = qs_ref[...] + c_ref[...]
    z = jnp.where(z > 0, z, 0.01 * z)
    e = jnp.exp(z)
    den_ref[pl.ds(lo8, W), :] += _pt_dot(p, e)
    s_ref[pl.ds(lo8, W), :] += _pt_dot(p, e * x)

    @pl.when(overflow)
    def _():
        pf = _onehot(seg_t) * (rel >= W).astype(jnp.bfloat16)
        den_ref[...] += _pt_dot(pf, e)
        s_ref[...] += _pt_dot(pf, e * x)


def _q_body(h_ref, w1a_ref, b1_ref, qb_ref):
    h = h_ref[...]
    qb_ref[...] = jnp.sum(jnp.maximum(h, 0.0) * w1a_ref[...], axis=1,
                          keepdims=True) + b1_ref[...]


def _dot3(a, b):
    ah, al = _split(a)
    bh, bl = _split(b)
    dn = (((1,), (1,)), ((), ()))
    return (lax.dot_general(ah, bh, dn, preferred_element_type=jnp.float32) +
            lax.dot_general(ah, bl, dn, preferred_element_type=jnp.float32) +
            lax.dot_general(al, bh, dn, preferred_element_type=jnp.float32))


def _update_body(s_ref, den_ref, h_ref, wp_ref, bp_ref, wih_ref, whh_ref,
                 bih_ref, bhh_ref, w1a_ref, b1_ref, hn_ref, qb_ref):
    den = den_ref[...]
    mask = den > 0
    sn = s_ref[...] * jnp.where(mask, 1.0 / jnp.where(mask, den, 1.0), 0.0)
    g = _dot3(sn, wp_ref[...]) + bp_ref[...]
    g = jnp.where(mask, g, 0.0)
    ctx = jnp.where(g > 0, g, jnp.exp(jnp.minimum(g, 0.0)) - 1.0)
    h = h_ref[...]
    gi = _dot3(ctx, wih_ref[...]) + bih_ref[...]
    gh = _dot3(h, whh_ref[...]) + bhh_ref[...]
    r = jax.nn.sigmoid(gi[:, :F] + gh[:, :F])
    zg = jax.nn.sigmoid(gi[:, F:2 * F] + gh[:, F:2 * F])
    n = jnp.tanh(gi[:, 2 * F:] + r * gh[:, 2 * F:])
    hn = (1.0 - zg) * n + zg * h
    hn_ref[...] = hn
    qb_ref[...] = jnp.sum(jnp.maximum(hn, 0.0) * w1a_ref[...], axis=1,
                          keepdims=True) + b1_ref[...]


def kernel(node_feats, weight, segment_ids, W1_0, b1_0, Wp_0, bp_0, Wih_0,
           Whh_0, bih_0, bhh_0, W1_1, b1_1, Wp_1, bp_1, Wih_1, Whh_1, bih_1,
           bhh_1):
    vp = ((V + R - 1) // R) * R
    nb = vp // R
    pad = vp - V
    x = jnp.pad(node_feats, ((0, pad), (0, 0)))
    w = jnp.pad(weight, (0, pad)).reshape(vp, 1)
    seg = jnp.pad(segment_ids.astype(jnp.int32), (0, pad),
                  constant_values=SG).reshape(vp, 1)

    xspec = pl.BlockSpec((R, F), lambda i: (i, 0))
    vspec = pl.BlockSpec((R, 1), lambda i: (i, 0))
    sspec = pl.BlockSpec((SG, F), lambda i: (0, 0))
    dspec = pl.BlockSpec((SG, 1), lambda i: (0, 0))

    seg_first = seg[::R].reshape(nb, 1, 1)
    fspec = pl.BlockSpec((1, 1, 1), lambda i: (i, 0, 0),
                         memory_space=pltpu.SMEM)

    w1b_both = jnp.concatenate([W1_0[:, F:].reshape(F, 1),
                                W1_1[:, F:].reshape(F, 1)], axis=1)
    s0, cc = pl.pallas_call(
        _seg_sum_w_body,
        grid=(nb,),
        in_specs=[fspec, xspec, vspec, vspec,
                  pl.BlockSpec((F, 2), lambda i: (0, 0))],
        out_specs=[sspec, pl.BlockSpec((R, 2), lambda i: (i, 0))],
        out_shape=[jax.ShapeDtypeStruct((SG, F), jnp.float32),
                   jax.ShapeDtypeStruct((vp, 2), jnp.float32)],
    )(seg_first, x, w, seg, w1b_both)
    c0 = cc[:, 0:1]
    c1 = cc[:, 1:2]

    SGB = min(512, SG)
    rowspec = pl.BlockSpec((SGB, F), lambda i: (i, 0))
    rvspec = pl.BlockSpec((SGB, 1), lambda i: (i, 0))
    full = lambda a, b: pl.BlockSpec(a, lambda i: b)

    def q_of(h, w1a, b1):
        return pl.pallas_call(
            _q_body,
            grid=(SG // SGB,),
            in_specs=[rowspec, full((1, F), (0, 0)), full((1, 1), (0, 0))],
            out_specs=rvspec,
            out_shape=jax.ShapeDtypeStruct((SG, 1), jnp.float32),
        )(h, w1a, b1)

    def attn(qb, cv):
        return pl.pallas_call(
            _seg_attn_body,
            grid=(nb,),
            in_specs=[fspec, xspec, vspec, full((SG, 1), (0, 0)),
                      vspec],
            out_specs=[sspec, dspec],
            out_shape=[jax.ShapeDtypeStruct((SG, F), jnp.float32),
                       jax.ShapeDtypeStruct((SG, 1), jnp.float32)],
            scratch_shapes=[pltpu.VMEM((R, 1), jnp.float32)],
        )(seg_first, x, seg, qb, cv)

    def update(s, den, h, Wp, bp, Wih, Whh, bih, bhh, w1a_n, b1_n):
        return pl.pallas_call(
            _update_body,
            grid=(SG // SGB,),
            in_specs=[rowspec, rvspec, rowspec,
                      full((F, F), (0, 0)), full((1, F), (0, 0)),
                      full((3 * F, F), (0, 0)), full((3 * F, F), (0, 0)),
                      full((1, 3 * F), (0, 0)), full((1, 3 * F), (0, 0)),
                      full((1, F), (0, 0)), full((1, 1), (0, 0))],
            out_specs=[rowspec, rvspec],
            out_shape=[jax.ShapeDtypeStruct((SG, F), jnp.float32),
                       jax.ShapeDtypeStruct((SG, 1), jnp.float32)],
        )(s, den, h, Wp, bp, Wih, Whh, bih, bhh, w1a_n, b1_n)

    w1a_0 = W1_0[:, :F]
    w1b_0 = W1_0[:, F:].reshape(F, 1)
    w1a_1 = W1_1[:, :F]
    w1b_1 = W1_1[:, F:].reshape(F, 1)
    b1_0r = b1_0.reshape(1, 1)
    b1_1r = b1_1.reshape(1, 1)
    bp_0r = bp_0.reshape(1, F)
    bp_1r = bp_1.reshape(1, F)
    bih_0r = bih_0.reshape(1, 3 * F)
    bhh_0r = bhh_0.reshape(1, 3 * F)
    bih_1r = bih_1.reshape(1, 3 * F)
    bhh_1r = bhh_1.reshape(1, 3 * F)

    qb0 = q_of(s0, w1a_0, b1_0r)
    s_a, den_a = attn(qb0, c0)
    h1, qb1 = update(s_a, den_a, s0, Wp_0, bp_0r, Wih_0, Whh_0, bih_0r,
                     bhh_0r, w1a_1, b1_1r)
    s_b, den_b = attn(qb1, c1)
    h2, _ = update(s_b, den_b, h1, Wp_1, bp_1r, Wih_1, Whh_1, bih_1r,
                   bhh_1r, w1a_1, b1_1r)
    return h2


# speedup vs baseline: 7.1186x; 1.1142x over previous
import functools
import jax
import jax.numpy as jnp
from jax import lax
from jax.experimental import pallas as pl
from jax.experimental.pallas import tpu as pltpu

V = 100000
F = 512
SG = 2048
R = 2048


W = 128


def _onehot(seg_t):
    cols = lax.broadcasted_iota(jnp.int32, (R, SG), 1)
    return (seg_t == cols).astype(jnp.bfloat16)


def _onehot_win(rel):
    cols = lax.broadcasted_iota(jnp.int32, (R, W), 1)
    return (rel == cols).astype(jnp.bfloat16)


def _split(y):
    hi = y.astype(jnp.bfloat16)
    lo = (y - hi.astype(jnp.float32)).astype(jnp.bfloat16)
    return hi, lo


def _pt_dot(p, y):
    hi, lo = _split(y)
    dn = (((0,), (0,)), ((), ()))
    return (lax.dot_general(p, hi, dn, preferred_element_type=jnp.float32) +
            lax.dot_general(p, lo, dn, preferred_element_type=jnp.float32))


def _seg_sum_w_body(first_ref, x_ref, w_ref, seg_ref, w1b_ref, s_ref,
                    cc_ref):
    i = pl.program_id(0)

    @pl.when(i == 0)
    def _():
        s_ref[...] = jnp.zeros_like(s_ref)

    seg_t = seg_ref[...]
    lo8 = jnp.minimum((first_ref[0, 0, 0] // 8) * 8, SG - W)
    rel = seg_t - lo8
    p = _onehot_win(rel)
    x = x_ref[...]
    xh, xl = _split(x)
    wbh, wbl = _split(w1b_ref[...])
    cc_ref[...] = (jnp.dot(xh, wbh, preferred_element_type=jnp.float32) +
                   jnp.dot(xh, wbl, preferred_element_type=jnp.float32) +
                   jnp.dot(xl, wbh, preferred_element_type=jnp.float32))
    y = w_ref[...] * x
    s_ref[pl.ds(lo8, W), :] += _pt_dot(p, y)

    @pl.when(jnp.max(rel) >= W)
    def _():
        pf = _onehot(seg_t) * (rel >= W).astype(jnp.bfloat16)
        s_ref[...] += _pt_dot(pf, y)


def _seg_attn_body(first_ref, x_ref, seg_ref, qb_ref, c_ref, s_ref,
                   den_ref, qs_ref):
    i = pl.program_id(0)

    @pl.when(i == 0)
    def _():
        s_ref[...] = jnp.zeros_like(s_ref)
        den_ref[...] = jnp.zeros_like(den_ref)

    seg_t = seg_ref[...]
    lo8 = jnp.minimum((first_ref[0, 0, 0] // 8) * 8, SG - W)
    rel = seg_t - lo8
    p = _onehot_win(rel)
    x = x_ref[...]
    overflow = jnp.max(rel) >= W

    qh, ql = _split(qb_ref[pl.ds(lo8, W), :])
    qs_ref[...] = (jnp.dot(p, qh, preferred_element_type=jnp.float32) +
                   jnp.dot(p, ql, preferred_element_type=jnp.float32))

    @pl.when(overflow)
    def _():
        pf = _onehot(seg_t) * (rel >= W).astype(jnp.bfloat16)
        qfh, qfl = _split(qb_ref[...])
        qs_ref[...] += (jnp.dot(pf, qfh, preferred_element_type=jnp.float32)
                        + jnp.dot(pf, qfl,
                                  preferred_element_type=jnp.float32))

    z = qs_ref[...] + c_ref[...]
    z = jnp.where(z > 0, z, 0.01 * z)
    e = jnp.exp(z)
    den_ref[pl.ds(lo8, W), :] += _pt_dot(p, e)
    s_ref[pl.ds(lo8, W), :] += _pt_dot(p, e * x)

    @pl.when(overflow)
    def _():
        pf = _onehot(seg_t) * (rel >= W).astype(jnp.bfloat16)
        den_ref[...] += _pt_dot(pf, e)
        s_ref[...] += _pt_dot(pf, e * x)


def _q_body(h_ref, w1a_ref, b1_ref, qb_ref):
    h = h_ref[...]
    qb_ref[...] = jnp.sum(jnp.maximum(h, 0.0) * w1a_ref[...], axis=1,
                          keepdims=True) + b1_ref[...]


def _dot3(a, b):
    ah, al = _split(a)
    bh, bl = _split(b)
    dn = (((1,), (1,)), ((), ()))
    return (lax.dot_general(ah, bh, dn, preferred_element_type=jnp.float32) +
            lax.dot_general(ah, bl, dn, preferred_element_type=jnp.float32) +
            lax.dot_general(al, bh, dn, preferred_element_type=jnp.float32))


def _update_body(s_ref, den_ref, h_ref, wp_ref, bp_ref, wih_ref, whh_ref,
                 bih_ref, bhh_ref, w1a_ref, b1_ref, hn_ref, qb_ref):
    den = den_ref[...]
    mask = den > 0
    sn = s_ref[...] * jnp.where(mask, 1.0 / jnp.where(mask, den, 1.0), 0.0)
    g = _dot3(sn, wp_ref[...]) + bp_ref[...]
    g = jnp.where(mask, g, 0.0)
    ctx = jnp.where(g > 0, g, jnp.exp(jnp.minimum(g, 0.0)) - 1.0)
    h = h_ref[...]
    gi = _dot3(ctx, wih_ref[...]) + bih_ref[...]
    gh = _dot3(h, whh_ref[...]) + bhh_ref[...]
    r = jax.nn.sigmoid(gi[:, :F] + gh[:, :F])
    zg = jax.nn.sigmoid(gi[:, F:2 * F] + gh[:, F:2 * F])
    n = jnp.tanh(gi[:, 2 * F:] + r * gh[:, 2 * F:])
    hn = (1.0 - zg) * n + zg * h
    hn_ref[...] = hn
    qb_ref[...] = jnp.sum(jnp.maximum(hn, 0.0) * w1a_ref[...], axis=1,
                          keepdims=True) + b1_ref[...]


def kernel(node_feats, weight, segment_ids, W1_0, b1_0, Wp_0, bp_0, Wih_0,
           Whh_0, bih_0, bhh_0, W1_1, b1_1, Wp_1, bp_1, Wih_1, Whh_1, bih_1,
           bhh_1):
    vp = ((V + R - 1) // R) * R
    nb = vp // R
    pad = vp - V
    x = jnp.pad(node_feats, ((0, pad), (0, 0)))
    w = jnp.pad(weight, (0, pad)).reshape(vp, 1)
    seg = jnp.pad(segment_ids.astype(jnp.int32), (0, pad),
                  constant_values=SG).reshape(vp, 1)

    xspec = pl.BlockSpec((R, F), lambda i: (i, 0))
    vspec = pl.BlockSpec((R, 1), lambda i: (i, 0))
    sspec = pl.BlockSpec((SG, F), lambda i: (0, 0))
    dspec = pl.BlockSpec((SG, 1), lambda i: (0, 0))

    seg_first = seg[::R].reshape(nb, 1, 1)
    fspec = pl.BlockSpec((1, 1, 1), lambda i: (i, 0, 0),
                         memory_space=pltpu.SMEM)

    w1b_both = jnp.concatenate([W1_0[:, F:].reshape(F, 1),
                                W1_1[:, F:].reshape(F, 1)], axis=1)
    s0, cc = pl.pallas_call(
        _seg_sum_w_body,
        grid=(nb,),
        in_specs=[fspec, xspec, vspec, vspec,
                  pl.BlockSpec((F, 2), lambda i: (0, 0))],
        out_specs=[sspec, pl.BlockSpec((R, 2), lambda i: (i, 0))],
        out_shape=[jax.ShapeDtypeStruct((SG, F), jnp.float32),
                   jax.ShapeDtypeStruct((vp, 2), jnp.float32)],
    )(seg_first, x, w, seg, w1b_both)
    c0 = cc[:, 0:1]
    c1 = cc[:, 1:2]

    SGB = min(512, SG)
    rowspec = pl.BlockSpec((SGB, F), lambda i: (i, 0))
    rvspec = pl.BlockSpec((SGB, 1), lambda i: (i, 0))
    full = lambda a, b: pl.BlockSpec(a, lambda i: b)

    def q_of(h, w1a, b1):
        return pl.pallas_call(
            _q_body,
            grid=(SG // SGB,),
            in_specs=[rowspec, full((1, F), (0, 0)), full((1, 1), (0, 0))],
            out_specs=rvspec,
            out_shape=jax.ShapeDtypeStruct((SG, 1), jnp.float32),
        )(h, w1a, b1)

    def attn(qb, cv):
        return pl.pallas_call(
            _seg_attn_body,
            grid=(nb,),
            in_specs=[fspec, xspec, vspec, full((SG, 1), (0, 0)),
                      vspec],
            out_specs=[sspec, dspec],
            out_shape=[jax.ShapeDtypeStruct((SG, F), jnp.float32),
                       jax.ShapeDtypeStruct((SG, 1), jnp.float32)],
            scratch_shapes=[pltpu.VMEM((R, 1), jnp.float32)],
        )(seg_first, x, seg, qb, cv)

    def update(s, den, h, Wp, bp, Wih, Whh, bih, bhh, w1a_n, b1_n):
        return pl.pallas_call(
            _update_body,
            grid=(SG // SGB,),
            in_specs=[rowspec, rvspec, rowspec,
                      full((F, F), (0, 0)), full((1, F), (0, 0)),
                      full((3 * F, F), (0, 0)), full((3 * F, F), (0, 0)),
                      full((1, 3 * F), (0, 0)), full((1, 3 * F), (0, 0)),
                      full((1, F), (0, 0)), full((1, 1), (0, 0))],
            out_specs=[rowspec, rvspec],
            out_shape=[jax.ShapeDtypeStruct((SG, F), jnp.float32),
                       jax.ShapeDtypeStruct((SG, 1), jnp.float32)],
        )(s, den, h, Wp, bp, Wih, Whh, bih, bhh, w1a_n, b1_n)

    w1a_0 = W1_0[:, :F]
    w1b_0 = W1_0[:, F:].reshape(F, 1)
    w1a_1 = W1_1[:, :F]
    w1b_1 = W1_1[:, F:].reshape(F, 1)
    b1_0r = b1_0.reshape(1, 1)
    b1_1r = b1_1.reshape(1, 1)
    bp_0r = bp_0.reshape(1, F)
    bp_1r = bp_1.reshape(1, F)
    bih_0r = bih_0.reshape(1, 3 * F)
    bhh_0r = bhh_0.reshape(1, 3 * F)
    bih_1r = bih_1.reshape(1, 3 * F)
    bhh_1r = bhh_1.reshape(1, 3 * F)

    qb0 = q_of(s0, w1a_0, b1_0r)
    s_a, den_a = attn(qb0, c0)
    h1, qb1 = update(s_a, den_a, s0, Wp_0, bp_0r, Wih_0, Whh_0, bih_0r,
                     bhh_0r, w1a_1, b1_1r)
    s_b, den_b = attn(qb1, c1)
    h2, _ = update(s_b, den_b, h1, Wp_1, bp_1r, Wih_1, Whh_1, bih_1r,
                   bhh_1r, w1a_1, b1_1r)
    return h2
